# baseline (speedup 1.0000x reference)
import functools

import numpy as np
import jax
import jax.numpy as jnp
from jax import lax
from jax.experimental import pallas as pl
from jax.experimental.pallas import tpu as pltpu
from jax.experimental.pallas import tpu_sc as plsc

COL = 16
NBLK = 64
L = COL * NBLK
INIT_W = 1024
DATA_ROWS = 4096
BATCH = 2048
NCHAIN = 1
BH = BATCH // NCHAIN
BT = 1024
NC, NS = 2, 16
NW = NC * NS

f32 = jnp.float32
i32 = jnp.int32

_QR, _SGN = [], []
for _li in range(6):
    _s = 2 ** (_li % 4)
    _ls = int(np.log2(_s))
    _p = np.arange(COL)
    _QR.append(((_p >> (_ls + 1)) * _s + (_p & (_s - 1))).astype(np.int32))
    _SGN.append(np.where(((_p >> _ls) & 1) == 0, 1.0, -1.0).astype(np.float32))


def _signed_angle_tables(angles_all):
    per_mod = []
    for m in range(4):
        rows = []
        for li in range(6):
            a = angles_all[m, li].reshape(NBLK, 8)[:, _QR[li]]
            rows.append(a.T * _SGN[li][:, None])
        per_mod.append(jnp.stack(rows))
    return jnp.stack(per_mod)


_MESH = plsc.VectorSubcoreMesh(
    core_axis_name="c", subcore_axis_name="s", num_cores=NC, num_subcores=NS)


def _wid():
    return lax.axis_index("s") * NC + lax.axis_index("c")


def _pm_positions(w, rows_per_worker, chunk, nblk_mod, blk_off):
    r0 = w * rows_per_worker + 16 * chunk
    lnb = int(np.log2(nblk_mod))
    p = lax.shift_right_logical(r0, lnb)
    b0 = lax.bitwise_and(r0, nblk_mod - 1)
    return (lax.iota(i32, 16) + (b0 + blk_off)) * COL + p


@functools.partial(
    pl.kernel, mesh=_MESH,
    out_type=jax.ShapeDtypeStruct((DATA_ROWS, BH), f32),
    scratch_types=[pltpu.VMEM((16, BH), f32), pltpu.VMEM((16,), i32),
                   pltpu.SemaphoreType.DMA],
    name="sc_init_scatter")
def _sc_init(scaled_hbm, zrows_hbm, data_hbm, rows_v, idx_v, sem):
    w = _wid()
    base = w * 16
    pltpu.sync_copy(zrows_hbm, rows_v)
    z = 2 * (lax.iota(i32, 16) + base) + 1
    idx_v[...] = z - lax.shift_right_logical(z + 1, 10)
    pltpu.async_copy(rows_v, data_hbm.at[idx_v], sem).wait()
    pltpu.sync_copy(scaled_hbm.at[pl.ds(base, 16)], rows_v)
    kk = lax.iota(i32, 16) + base
    idx_v[...] = 2 * kk + lax.shift_right_logical(kk + 1, 9)
    pltpu.async_copy(rows_v, data_hbm.at[idx_v], sem).wait()


def _make_sc_gather(rows_total):
    r = rows_total // NW

    @functools.partial(
        pl.kernel, mesh=_MESH,
        out_type=jax.ShapeDtypeStruct((rows_total, BH), f32),
        scratch_types=[pltpu.VMEM((r, BH), f32), pltpu.VMEM((r,), i32),
                       pltpu.SemaphoreType.DMA],
        name=f"sc_gather_{rows_total}")
    def k(gidx_hbm, data_ref, out_hbm, rows_v, idx_v, sem):
        base = _wid() * r
        pltpu.sync_copy(gidx_hbm.at[pl.ds(base, r)], idx_v)
        pltpu.async_copy(data_ref.at[idx_v], rows_v, sem).wait()
        pltpu.sync_copy(rows_v, out_hbm.at[pl.ds(base, r)])

    return k


_sc_gather_1024 = _make_sc_gather(L)
_sc_gather_512 = _make_sc_gather(512)


def _make_sc_scatter(idx_out):
    @functools.partial(
        pl.kernel, mesh=_MESH, out_type=(),
        scratch_types=[pltpu.VMEM((32, BH), f32), pltpu.VMEM((32,), i32),
                       pltpu.SemaphoreType.DMA],
        name=f"sc_scatter_{idx_out}")
    def k(xb_hbm, act_hbm, gidx_hbm, data_ref, rows_v, idx_v, sem):
        w = _wid()
        base = w * 32
        pltpu.sync_copy(act_hbm.at[pl.ds(base, 32)],
                        data_ref.at[pl.ds(idx_out + base, 32)])
        pltpu.sync_copy(xb_hbm.at[pl.ds(base, 32)], rows_v)
        pltpu.sync_copy(gidx_hbm.at[pl.ds(base, 32)], idx_v)
        pltpu.async_copy(rows_v, data_ref.at[idx_v], sem).wait()

    return k


_sc_scatters = [_make_sc_scatter(INIT_W * (i + 1)) for i in range(3)]


def _remap_pm(v):
    r = jnp.bitwise_and(v, 1023)
    pm = (v & ~1023) + jnp.bitwise_and(r, 15) * NBLK + (r >> 4)
    return jnp.where(v >= INIT_W, pm, v)


@functools.partial(
    pl.kernel, mesh=_MESH,
    out_type=jax.ShapeDtypeStruct((512, BH), f32),
    scratch_types=[pltpu.VMEM((16, BH), f32), pltpu.VMEM((16,), i32),
                   pltpu.SemaphoreType.DMA],
    name="sc_scatter_out")
def _sc_scatter_out(act_hbm, out_hbm, rows_v, idx_v, sem):
    w = _wid()
    pltpu.sync_copy(act_hbm.at[pl.ds(w * 16, 16)], rows_v)
    idx_v[...] = _pm_positions(w, 16, 0, 32, 0)
    pltpu.async_copy(rows_v, out_hbm.at[idx_v], sem).wait()


def _prescale_and_tables(input_data, scales, ang_flat):
    def body(x_ref, s_ref, a_ref, o_ref, c_ref, sn_ref):
        o_ref[...] = x_ref[...] * s_ref[...]
        a = a_ref[...]
        c_ref[...] = jnp.cos(a)
        sn_ref[...] = jnp.sin(a)

    ts = jax.ShapeDtypeStruct((384, NBLK), f32)
    tspec = pl.BlockSpec((384, NBLK), lambda j: (0, 0))
    return pl.pallas_call(
        body,
        grid=(BH // BT,),
        in_specs=[pl.BlockSpec((512, BT), lambda j: (0, j)),
                  pl.BlockSpec((512, 1), lambda j: (0, 0)),
                  tspec],
        out_specs=[pl.BlockSpec((512, BT), lambda j: (0, j)), tspec, tspec],
        out_shape=[jax.ShapeDtypeStruct((512, BH), f32), ts, ts],
    )(input_data, scales[:, None], ang_flat)


def _rotate(x, c, s, li, nb):
    st = 2 ** (li % 4)
    xr = x.reshape(COL // (2 * st), 2, st, nb, BT)
    part = jnp.concatenate([xr[:, 1:2], xr[:, 0:1]], axis=1)
    part = part.reshape(COL, nb, BT)
    return c * x + s * part


def _tc_module_full(g, ctab, stab, btab):
    def body(g_ref, c_ref, s_ref, b_ref, xb_ref, act_ref):
        call = c_ref[...].reshape(6, COL, NBLK)[..., None]
        sall = s_ref[...].reshape(6, COL, NBLK)[..., None]
        x = g_ref[...].reshape(COL, NBLK, BT)
        for li in range(3):
            x = _rotate(x, call[li], sall[li], li, NBLK)
        y = x + b_ref[...][..., None]
        act_ref[...] = (0.5 * (y + jnp.sqrt(y * y + 1.0))).reshape(L, BT)
        for li in range(3, 6):
            x = _rotate(x, call[li], sall[li], li, NBLK)
        xb_ref[...] = x.reshape(L, BT)

    os = jax.ShapeDtypeStruct((L, BH), f32)
    tspec = pl.BlockSpec((96, NBLK), lambda j: (0, 0))
    return pl.pallas_call(
        body,
        grid=(BH // BT,),
        in_specs=[pl.BlockSpec((L, BT), lambda j: (0, j)),
                  tspec, tspec,
                  pl.BlockSpec((COL, NBLK), lambda j: (0, 0))],
        out_specs=[pl.BlockSpec((L, BT), lambda j: (0, j)),
                   pl.BlockSpec((L, BT), lambda j: (0, j))],
        out_shape=[os, os],
    )(g, ctab, stab, btab)


def _tc_module_final(g, ctab, stab, btab):
    nb = 32

    def body(g_ref, c_ref, s_ref, b_ref, act_ref):
        call = c_ref[...].reshape(3, COL, nb)[..., None]
        sall = s_ref[...].reshape(3, COL, nb)[..., None]
        x = g_ref[...].reshape(COL, nb, BT)
        for li in range(3):
            x = _rotate(x, call[li], sall[li], li, nb)
        y = x + b_ref[...][..., None]
        act_ref[...] = (0.5 * (y + jnp.sqrt(y * y + 1.0))).reshape(512, BT)

    tspec = pl.BlockSpec((48, nb), lambda j: (0, 0))
    return pl.pallas_call(
        body,
        grid=(BH // BT,),
        in_specs=[pl.BlockSpec((512, BT), lambda j: (0, j)),
                  tspec, tspec,
                  pl.BlockSpec((COL, nb), lambda j: (0, 0))],
        out_specs=pl.BlockSpec((512, BT), lambda j: (0, j)),
        out_shape=jax.ShapeDtypeStruct((512, BH), f32),
    )(g, ctab, stab, btab)


def kernel(input_data, scales, angles_all, biases_all, indices_all):
    ang = _signed_angle_tables(angles_all).reshape(4 * 6 * COL, NBLK)
    scaled, cflat, sflat = _prescale_and_tables(input_data, scales, ang)
    ct = [cflat[96 * i:96 * (i + 1)] for i in range(4)]
    st = [sflat[96 * i:96 * (i + 1)] for i in range(4)]
    bias_t = [biases_all[i].reshape(NBLK, COL).T for i in range(4)]
    gidx_pm = _remap_pm(jnp.swapaxes(indices_all.reshape(4, NBLK, COL), 1, 2))

    zrows = jnp.zeros((16, BH), f32)
    data = jax.new_ref(_sc_init(scaled, zrows))

    for i in range(3):
        gidx = gidx_pm[i].reshape(L)
        g = _sc_gather_1024(gidx, data)
        xb, act = _tc_module_full(g, ct[i], st[i], bias_t[i])
        _sc_scatters[i](xb, act, gidx, data)

    gidx3 = gidx_pm[3, :, 32:].reshape(512)
    g3 = _sc_gather_512(gidx3, data)
    act3 = _tc_module_final(g3, ct[3][:48, 32:], st[3][:48, 32:],
                            bias_t[3][:, 32:])
    return _sc_scatter_out(act3)


# speedup vs baseline: 4.8035x; 4.8035x over previous
import functools

import numpy as np
import jax
import jax.numpy as jnp
from jax import lax
from jax.experimental import pallas as pl
from jax.experimental.pallas import tpu as pltpu
from jax.experimental.pallas import tpu_sc as plsc

COL = 16
NBLK = 64
L = COL * NBLK
INIT_W = 1024
DATA_ROWS = 4096
BATCH = 2048
NCHAIN = 1
BH = BATCH // NCHAIN
BT = 1024
NC, NS = 2, 16
NW = NC * NS

f32 = jnp.float32
i32 = jnp.int32

_QR, _SGN = [], []
for _li in range(6):
    _s = 2 ** (_li % 4)
    _ls = int(np.log2(_s))
    _p = np.arange(COL)
    _QR.append(((_p >> (_ls + 1)) * _s + (_p & (_s - 1))).astype(np.int32))
    _SGN.append(np.where(((_p >> _ls) & 1) == 0, 1.0, -1.0).astype(np.float32))


def _signed_angle_tables(angles_all):
    per_mod = []
    for m in range(4):
        rows = []
        for li in range(6):
            a = angles_all[m, li].reshape(NBLK, 8)[:, _QR[li]]
            rows.append(a.T * _SGN[li][:, None])
        per_mod.append(jnp.stack(rows))
    return jnp.stack(per_mod)


_MESH = plsc.VectorSubcoreMesh(
    core_axis_name="c", subcore_axis_name="s", num_cores=NC, num_subcores=NS)


def _wid():
    return lax.axis_index("s") * NC + lax.axis_index("c")


def _pm_positions(w, rows_per_worker, chunk, nblk_mod, blk_off):
    r0 = w * rows_per_worker + 16 * chunk
    lnb = int(np.log2(nblk_mod))
    p = lax.shift_right_logical(r0, lnb)
    b0 = lax.bitwise_and(r0, nblk_mod - 1)
    return (lax.iota(i32, 16) + (b0 + blk_off)) * COL + p


@functools.partial(
    pl.kernel, mesh=_MESH,
    out_type=jax.ShapeDtypeStruct((DATA_ROWS, BH), f32),
    scratch_types=[pltpu.VMEM((16, BH), f32), pltpu.VMEM((16,), i32),
                   pltpu.SemaphoreType.DMA],
    name="sc_init_scatter")
def _sc_init(scaled_hbm, zrows_hbm, data_hbm, rows_v, idx_v, sem):
    w = _wid()
    base = w * 16
    pltpu.sync_copy(zrows_hbm, rows_v)
    z = 2 * (lax.iota(i32, 16) + base) + 1
    idx_v[...] = z - lax.shift_right_logical(z + 1, 10)
    pltpu.async_copy(rows_v, data_hbm.at[idx_v], sem).wait()
    pltpu.sync_copy(scaled_hbm.at[pl.ds(base, 16)], rows_v)
    kk = lax.iota(i32, 16) + base
    idx_v[...] = 2 * kk + lax.shift_right_logical(kk + 1, 9)
    pltpu.async_copy(rows_v, data_hbm.at[idx_v], sem).wait()


def _make_sc_gather(rows_total):
    r = rows_total // NW

    @functools.partial(
        pl.kernel, mesh=_MESH,
        out_type=jax.ShapeDtypeStruct((rows_total, BH), f32),
        scratch_types=[pltpu.VMEM((r, BH), f32), pltpu.VMEM((r,), i32),
                       pltpu.SemaphoreType.DMA],
        name=f"sc_gather_{rows_total}")
    def k(gidx_hbm, data_ref, out_hbm, rows_v, idx_v, sem):
        base = _wid() * r
        pltpu.sync_copy(gidx_hbm.at[pl.ds(base, r)], idx_v)
        pltpu.async_copy(data_ref.at[idx_v], rows_v, sem).wait()
        pltpu.sync_copy(rows_v, out_hbm.at[pl.ds(base, r)])

    return k


_sc_gather_1024 = _make_sc_gather(L)
_sc_gather_512 = _make_sc_gather(512)


def _make_sc_scatter(idx_out):
    @functools.partial(
        pl.kernel, mesh=_MESH, out_type=(),
        scratch_types=[pltpu.VMEM((32, BH), f32), pltpu.VMEM((32,), i32),
                       pltpu.SemaphoreType.DMA],
        name=f"sc_scatter_{idx_out}")
    def k(xb_hbm, act_hbm, gidx_hbm, data_ref, rows_v, idx_v, sem):
        w = _wid()
        base = w * 32
        pltpu.sync_copy(xb_hbm.at[pl.ds(base, 32)], rows_v)
        pltpu.sync_copy(gidx_hbm.at[pl.ds(base, 32)], idx_v)
        pltpu.async_copy(rows_v, data_ref.at[idx_v], sem).wait()
        pltpu.sync_copy(act_hbm.at[pl.ds(base, 32)], rows_v)
        pltpu.sync_copy(rows_v, data_ref.at[pl.ds(idx_out + base, 32)])

    return k


_sc_scatters = [_make_sc_scatter(INIT_W * (i + 1)) for i in range(3)]


def _remap_pm(v):
    r = jnp.bitwise_and(v, 1023)
    pm = (v & ~1023) + jnp.bitwise_and(r, 15) * NBLK + (r >> 4)
    return jnp.where(v >= INIT_W, pm, v)


@functools.partial(
    pl.kernel, mesh=_MESH,
    out_type=jax.ShapeDtypeStruct((512, BH), f32),
    scratch_types=[pltpu.VMEM((16, BH), f32), pltpu.VMEM((16,), i32),
                   pltpu.SemaphoreType.DMA],
    name="sc_scatter_out")
def _sc_scatter_out(act_hbm, out_hbm, rows_v, idx_v, sem):
    w = _wid()
    pltpu.sync_copy(act_hbm.at[pl.ds(w * 16, 16)], rows_v)
    idx_v[...] = _pm_positions(w, 16, 0, 32, 0)
    pltpu.async_copy(rows_v, out_hbm.at[idx_v], sem).wait()


def _prescale_and_tables(input_data, scales, ang_flat):
    def body(x_ref, s_ref, a_ref, o_ref, c_ref, sn_ref):
        o_ref[...] = x_ref[...] * s_ref[...]
        a = a_ref[...]
        c_ref[...] = jnp.cos(a)
        sn_ref[...] = jnp.sin(a)

    ts = jax.ShapeDtypeStruct((384, NBLK), f32)
    tspec = pl.BlockSpec((384, NBLK), lambda j: (0, 0))
    return pl.pallas_call(
        body,
        grid=(BH // BT,),
        in_specs=[pl.BlockSpec((512, BT), lambda j: (0, j)),
                  pl.BlockSpec((512, 1), lambda j: (0, 0)),
                  tspec],
        out_specs=[pl.BlockSpec((512, BT), lambda j: (0, j)), tspec, tspec],
        out_shape=[jax.ShapeDtypeStruct((512, BH), f32), ts, ts],
    )(input_data, scales[:, None], ang_flat)


def _rotate(x, c, s, li, nb):
    st = 2 ** (li % 4)
    xr = x.reshape(COL // (2 * st), 2, st, nb, BT)
    part = jnp.concatenate([xr[:, 1:2], xr[:, 0:1]], axis=1)
    part = part.reshape(COL, nb, BT)
    return c * x + s * part


def _tc_module_full(g, ctab, stab, btab):
    def body(g_ref, c_ref, s_ref, b_ref, xb_ref, act_ref):
        call = c_ref[...].reshape(6, COL, NBLK)[..., None]
        sall = s_ref[...].reshape(6, COL, NBLK)[..., None]
        x = g_ref[...].reshape(COL, NBLK, BT)
        for li in range(3):
            x = _rotate(x, call[li], sall[li], li, NBLK)
        y = x + b_ref[...][..., None]
        act_ref[...] = (0.5 * (y + jnp.sqrt(y * y + 1.0))).reshape(L, BT)
        for li in range(3, 6):
            x = _rotate(x, call[li], sall[li], li, NBLK)
        xb_ref[...] = x.reshape(L, BT)

    os = jax.ShapeDtypeStruct((L, BH), f32)
    tspec = pl.BlockSpec((96, NBLK), lambda j: (0, 0))
    return pl.pallas_call(
        body,
        grid=(BH // BT,),
        in_specs=[pl.BlockSpec((L, BT), lambda j: (0, j)),
                  tspec, tspec,
                  pl.BlockSpec((COL, NBLK), lambda j: (0, 0))],
        out_specs=[pl.BlockSpec((L, BT), lambda j: (0, j)),
                   pl.BlockSpec((L, BT), lambda j: (0, j))],
        out_shape=[os, os],
    )(g, ctab, stab, btab)


def _tc_module_final(g, ctab, stab, btab):
    nb = 32

    def body(g_ref, c_ref, s_ref, b_ref, act_ref):
        call = c_ref[...].reshape(3, COL, nb)[..., None]
        sall = s_ref[...].reshape(3, COL, nb)[..., None]
        x = g_ref[...].reshape(COL, nb, BT)
        for li in range(3):
            x = _rotate(x, call[li], sall[li], li, nb)
        y = x + b_ref[...][..., None]
        act_ref[...] = (0.5 * (y + jnp.sqrt(y * y + 1.0))).reshape(512, BT)

    tspec = pl.BlockSpec((48, nb), lambda j: (0, 0))
    return pl.pallas_call(
        body,
        grid=(BH // BT,),
        in_specs=[pl.BlockSpec((512, BT), lambda j: (0, j)),
                  tspec, tspec,
                  pl.BlockSpec((COL, nb), lambda j: (0, 0))],
        out_specs=pl.BlockSpec((512, BT), lambda j: (0, j)),
        out_shape=jax.ShapeDtypeStruct((512, BH), f32),
    )(g, ctab, stab, btab)


def kernel(input_data, scales, angles_all, biases_all, indices_all):
    ang = _signed_angle_tables(angles_all).reshape(4 * 6 * COL, NBLK)
    scaled, cflat, sflat = _prescale_and_tables(input_data, scales, ang)
    ct = [cflat[96 * i:96 * (i + 1)] for i in range(4)]
    st = [sflat[96 * i:96 * (i + 1)] for i in range(4)]
    bias_t = [biases_all[i].reshape(NBLK, COL).T for i in range(4)]
    gidx_pm = _remap_pm(jnp.swapaxes(indices_all.reshape(4, NBLK, COL), 1, 2))

    zrows = jnp.zeros((16, BH), f32)
    data = jax.new_ref(_sc_init(scaled, zrows))

    for i in range(3):
        gidx = gidx_pm[i].reshape(L)
        g = _sc_gather_1024(gidx, data)
        xb, act = _tc_module_full(g, ct[i], st[i], bias_t[i])
        _sc_scatters[i](xb, act, gidx, data)

    gidx3 = gidx_pm[3, :, 32:].reshape(512)
    g3 = _sc_gather_512(gidx3, data)
    act3 = _tc_module_final(g3, ct[3][:48, 32:], st[3][:48, 32:],
                            bias_t[3][:, 32:])
    return _sc_scatter_out(act3)
